# Initial kernel scaffold; baseline (speedup 1.0000x reference)
import jax
import jax.numpy as jnp
from jax.experimental import pallas as pl


def kernel(x, edge_index, W_in, b_in, W_pred, b_pred):
    raise NotImplementedError("write your pallas kernel here")


# speedup vs baseline: 33.7859x; 33.7859x over previous
import functools

import jax
import jax.numpy as jnp
from jax import lax
from jax.experimental import pallas as pl
from jax.experimental.pallas import tpu as pltpu
from jax.experimental.pallas import tpu_sc as plsc

N = 10000
E = 320000
D_IN = 128
D_ENC = 128
NP = 10240
NT = 16
EPT = E // NT
NPT = NP // NT
L = 16


def _tc_body(x_ref, w_ref, b_ref, wp_ref, o_ref):
    i = pl.program_id(0)
    h = jax.lax.dot_general(
        x_ref[...], w_ref[...], (((1,), (0,)), ((), ())),
        preferred_element_type=jnp.float32,
        precision=jax.lax.Precision.HIGHEST)
    h = jnp.maximum(h + b_ref[...], 0.0)
    v = jax.lax.dot_general(
        wp_ref[...], h, (((1,), (1,)), ((), ())),
        preferred_element_type=jnp.float32,
        precision=jax.lax.Precision.HIGHEST)
    row = i * 128 + jax.lax.broadcasted_iota(jnp.int32, (1, 1, 128), 2)
    o_ref[...] = jnp.where(row < N, v.reshape(1, 1, 128), 0.0)


def _tc_v0(xp, W_in, b2, wp2):
    return pl.pallas_call(
        _tc_body,
        grid=(NP // 128,),
        in_specs=[
            pl.BlockSpec((128, D_IN), lambda i: (i, 0)),
            pl.BlockSpec((D_IN, D_ENC), lambda i: (0, 0)),
            pl.BlockSpec((1, D_ENC), lambda i: (0, 0)),
            pl.BlockSpec((1, D_ENC), lambda i: (0, 0)),
        ],
        out_specs=pl.BlockSpec((1, 1, 128), lambda i: (i, 0, 0)),
        out_shape=jax.ShapeDtypeStruct((NP // 128, 1, 128), jnp.float32),
    )(xp, W_in, b2, wp2)


def _sc_body(v0_hbm, src_hbm, dst_hbm, bp_hbm,
             out_hbm, part_hbm, vcur_hbm,
             src_loc, dst_loc, v_loc, acc_loc, inv_loc, vnew_loc,
             red_loc, sum_loc, b_loc, out_loc):
    c = lax.axis_index("c")
    s = lax.axis_index("s")

    @pl.when(c == 0)
    def _main():
        wid = s
        ebase = wid * EPT
        nbase = wid * NPT

        pltpu.sync_copy(src_hbm.at[pl.ds(ebase, EPT)], src_loc)
        pltpu.sync_copy(dst_hbm.at[pl.ds(ebase, EPT)], dst_loc)
        pltpu.sync_copy(v0_hbm, v_loc)
        pltpu.sync_copy(bp_hbm, b_loc)

        zeros16 = jnp.zeros((L,), jnp.float32)
        ones16 = jnp.ones((L,), jnp.float32)

        def zero_acc():
            def zb(i, _):
                acc_loc[pl.ds(pl.multiple_of(i * L, L), L)] = zeros16
                return 0
            lax.fori_loop(0, NP // L, zb, 0)

        def edge_pass(count_only):
            def eb(i, _):
                b16 = pl.multiple_of(i * L, L)
                d16 = dst_loc[pl.ds(b16, L)]
                if count_only:
                    vals = ones16
                else:
                    s16 = src_loc[pl.ds(b16, L)]
                    vals = plsc.load_gather(v_loc, [s16])
                plsc.addupdate_scatter(acc_loc, [d16], vals)
                return 0
            lax.fori_loop(0, EPT // L, eb, 0)

        def combine():
            pltpu.sync_copy(acc_loc, part_hbm.at[wid])
            plsc.subcore_barrier()
            pltpu.sync_copy(part_hbm.at[:, pl.ds(nbase, NPT)], red_loc)

        def reduced(jb):
            tot = red_loc[0, pl.ds(jb, L)]
            for k in range(1, NT):
                tot = tot + red_loc[k, pl.ds(jb, L)]
            return tot

        zero_acc()
        edge_pass(True)
        combine()

        def invb(j, _):
            jb = pl.multiple_of(j * L, L)
            inv_loc[pl.ds(jb, L)] = 1.0 / jnp.maximum(reduced(jb), 1.0)
            return 0
        lax.fori_loop(0, NPT // L, invb, 0)
        plsc.subcore_barrier()

        for step in range(2):
            zero_acc()
            edge_pass(False)
            combine()

            def upd(j, _):
                jb = pl.multiple_of(j * L, L)
                m = reduced(jb) * inv_loc[pl.ds(jb, L)]
                vold = v_loc[pl.ds(pl.multiple_of(nbase + j * L, L), L)]
                vnew_loc[pl.ds(jb, L)] = (vold + m) * 0.5
                return 0
            lax.fori_loop(0, NPT // L, upd, 0)

            if step == 0:
                pltpu.sync_copy(vnew_loc, vcur_hbm.at[pl.ds(nbase, NPT)])
                plsc.subcore_barrier()
                pltpu.sync_copy(vcur_hbm, v_loc)
                plsc.subcore_barrier()

        def sb(j, a):
            return a + vnew_loc[pl.ds(pl.multiple_of(j * L, L), L)]
        acc16 = lax.fori_loop(0, NPT // L, sb, zeros16)
        plsc.subcore_barrier()
        out_loc[...] = acc16
        pltpu.sync_copy(out_loc, part_hbm.at[0, pl.ds(pl.multiple_of(wid * L, L), L)])
        plsc.subcore_barrier()

        @pl.when(s == 0)
        def _fin():
            pltpu.sync_copy(part_hbm.at[0, pl.ds(0, NT * L)], sum_loc)
            t = sum_loc[pl.ds(0, L)]
            for k in range(1, NT):
                t = t + sum_loc[pl.ds(k * L, L)]
            o = jnp.sum(t) * (1.0 / float(N))
            lane = lax.iota(jnp.int32, L)
            out_loc[...] = jnp.where(lane == 0, o, 0.0) + b_loc[...]
            pltpu.sync_copy(out_loc, out_hbm)


_sc_call = functools.partial(
    pl.kernel,
    mesh=plsc.VectorSubcoreMesh(core_axis_name="c", subcore_axis_name="s"),
    compiler_params=pltpu.CompilerParams(needs_layout_passes=False),
    out_type=(
        jax.ShapeDtypeStruct((L,), jnp.float32),
        jax.ShapeDtypeStruct((NT, NP), jnp.float32),
        jax.ShapeDtypeStruct((NP,), jnp.float32),
    ),
    scratch_types=[
        pltpu.VMEM((EPT,), jnp.int32),
        pltpu.VMEM((EPT,), jnp.int32),
        pltpu.VMEM((NP,), jnp.float32),
        pltpu.VMEM((NP,), jnp.float32),
        pltpu.VMEM((NPT,), jnp.float32),
        pltpu.VMEM((NPT,), jnp.float32),
        pltpu.VMEM((NT, NPT), jnp.float32),
        pltpu.VMEM((NT * L,), jnp.float32),
        pltpu.VMEM((L,), jnp.float32),
        pltpu.VMEM((L,), jnp.float32),
    ],
)(_sc_body)


def kernel(x, edge_index, W_in, b_in, W_pred, b_pred):
    xp = jnp.pad(x, ((0, NP - N), (0, 0)))
    b2 = b_in.reshape(1, D_ENC)
    wp2 = W_pred[:D_ENC, 0].reshape(1, D_ENC)
    v0 = _tc_v0(xp, W_in, b2, wp2).reshape(NP)
    src = edge_index[0]
    dst = edge_index[1]
    bp = jnp.pad(b_pred, (0, L - 1))
    out16, _, _ = _sc_call(v0, src, dst, bp)
    return out16[:1]


# speedup vs baseline: 43.0031x; 1.2728x over previous
import functools

import jax
import jax.numpy as jnp
from jax import lax
from jax.experimental import pallas as pl
from jax.experimental.pallas import tpu as pltpu
from jax.experimental.pallas import tpu_sc as plsc

N = 10000
E = 320000
D_IN = 128
D_ENC = 128
NB = 79
NV = NB * 128
NP = 10240
NT = 16
EPT = E // NT
NPT = NP // NT
L = 16


def _tc_body(x_ref, w_ref, b_ref, wp_ref, o_ref):
    i = pl.program_id(0)
    h = jax.lax.dot_general(
        x_ref[...], w_ref[...], (((1,), (0,)), ((), ())),
        preferred_element_type=jnp.float32,
        precision=jax.lax.Precision.HIGHEST)
    h = jnp.maximum(h + b_ref[...], 0.0)
    v = jax.lax.dot_general(
        wp_ref[...], h, (((1,), (1,)), ((), ())),
        preferred_element_type=jnp.float32,
        precision=jax.lax.Precision.HIGHEST)
    row = i * 128 + jax.lax.broadcasted_iota(jnp.int32, (1, 1, 128), 2)
    o_ref[...] = jnp.where(row < N, v.reshape(1, 1, 128), 0.0)


def _tc_v0(x, W_in, b2, wp2):
    return pl.pallas_call(
        _tc_body,
        grid=(NB,),
        in_specs=[
            pl.BlockSpec((128, D_IN), lambda i: (i, 0)),
            pl.BlockSpec((D_IN, D_ENC), lambda i: (0, 0)),
            pl.BlockSpec((1, D_ENC), lambda i: (0, 0)),
            pl.BlockSpec((1, D_ENC), lambda i: (0, 0)),
        ],
        out_specs=pl.BlockSpec((1, 1, 128), lambda i: (i, 0, 0)),
        out_shape=jax.ShapeDtypeStruct((NB, 1, 128), jnp.float32),
    )(x, W_in, b2, wp2)


def _sc_body(v0_hbm, src_hbm, dst_hbm, bp_hbm,
             out_hbm, part_hbm, vcur_hbm,
             src_loc, dst_loc, v_loc, acc_loc, inv_loc, vnew_loc,
             red_loc, sum_loc, b_loc, out_loc):
    c = lax.axis_index("c")
    s = lax.axis_index("s")

    @pl.when(c == 0)
    def _main():
        wid = s
        ebase = wid * EPT
        nbase = wid * NPT

        pltpu.sync_copy(src_hbm.at[pl.ds(ebase, EPT)], src_loc)
        pltpu.sync_copy(dst_hbm.at[pl.ds(ebase, EPT)], dst_loc)
        pltpu.sync_copy(v0_hbm, v_loc.at[pl.ds(0, NV)])
        pltpu.sync_copy(bp_hbm, b_loc)

        zeros16 = jnp.zeros((L,), jnp.float32)
        ones16 = jnp.ones((L,), jnp.float32)

        @plsc.parallel_loop(NV, NP, L)
        def _ztail(i):
            v_loc[pl.ds(i, L)] = zeros16

        def zero_acc():
            @plsc.parallel_loop(0, NP, L, unroll=8)
            def _zb(i):
                acc_loc[pl.ds(i, L)] = zeros16

        def edge_pass(count_only):
            @plsc.parallel_loop(0, EPT, L, unroll=8)
            def _eb(i):
                d16 = dst_loc[pl.ds(i, L)]
                if count_only:
                    vals = ones16
                else:
                    s16 = src_loc[pl.ds(i, L)]
                    vals = plsc.load_gather(v_loc, [s16])
                plsc.addupdate_scatter(acc_loc, [d16], vals)

        def combine():
            pltpu.sync_copy(acc_loc, part_hbm.at[wid])
            plsc.subcore_barrier()
            pltpu.sync_copy(part_hbm.at[:, pl.ds(nbase, NPT)], red_loc)

        def reduced(jb):
            tot = red_loc[0, pl.ds(jb, L)]
            for k in range(1, NT):
                tot = tot + red_loc[k, pl.ds(jb, L)]
            return tot

        zero_acc()
        edge_pass(True)
        combine()

        @plsc.parallel_loop(0, NPT, L, unroll=4)
        def _invb(jb):
            inv_loc[pl.ds(jb, L)] = 1.0 / jnp.maximum(reduced(jb), 1.0)

        plsc.subcore_barrier()

        for step in range(2):
            zero_acc()
            edge_pass(False)
            combine()

            @plsc.parallel_loop(0, NPT, L, unroll=4)
            def _upd(jb):
                m = reduced(jb) * inv_loc[pl.ds(jb, L)]
                vold = v_loc[pl.ds(pl.multiple_of(nbase + jb, L), L)]
                vnew_loc[pl.ds(jb, L)] = (vold + m) * 0.5

            if step == 0:
                pltpu.sync_copy(vnew_loc, vcur_hbm.at[pl.ds(nbase, NPT)])
                plsc.subcore_barrier()
                pltpu.sync_copy(vcur_hbm, v_loc)
                plsc.subcore_barrier()

        @plsc.parallel_loop(0, NPT, L, unroll=4, carry=zeros16)
        def acc16(jb, a):
            return a + vnew_loc[pl.ds(jb, L)]

        plsc.subcore_barrier()
        out_loc[...] = acc16
        pltpu.sync_copy(out_loc, part_hbm.at[0, pl.ds(pl.multiple_of(wid * L, L), L)])
        plsc.subcore_barrier()

        @pl.when(s == 0)
        def _fin():
            pltpu.sync_copy(part_hbm.at[0, pl.ds(0, NT * L)], sum_loc)
            t = sum_loc[pl.ds(0, L)]
            for k in range(1, NT):
                t = t + sum_loc[pl.ds(k * L, L)]
            o = jnp.sum(t) * (1.0 / float(N))
            lane = lax.iota(jnp.int32, L)
            out_loc[...] = jnp.where(lane == 0, o, 0.0) + b_loc[...]
            pltpu.sync_copy(out_loc, out_hbm)


_sc_call = functools.partial(
    pl.kernel,
    mesh=plsc.VectorSubcoreMesh(core_axis_name="c", subcore_axis_name="s"),
    compiler_params=pltpu.CompilerParams(needs_layout_passes=False),
    out_type=(
        jax.ShapeDtypeStruct((L,), jnp.float32),
        jax.ShapeDtypeStruct((NT, NP), jnp.float32),
        jax.ShapeDtypeStruct((NP,), jnp.float32),
    ),
    scratch_types=[
        pltpu.VMEM((EPT,), jnp.int32),
        pltpu.VMEM((EPT,), jnp.int32),
        pltpu.VMEM((NP,), jnp.float32),
        pltpu.VMEM((NP,), jnp.float32),
        pltpu.VMEM((NPT,), jnp.float32),
        pltpu.VMEM((NPT,), jnp.float32),
        pltpu.VMEM((NT, NPT), jnp.float32),
        pltpu.VMEM((NT * L,), jnp.float32),
        pltpu.VMEM((L,), jnp.float32),
        pltpu.VMEM((L,), jnp.float32),
    ],
)(_sc_body)


def kernel(x, edge_index, W_in, b_in, W_pred, b_pred):
    b2 = b_in.reshape(1, D_ENC)
    wp2 = W_pred[:D_ENC, 0].reshape(1, D_ENC)
    v0 = _tc_v0(x, W_in, b2, wp2).reshape(NV)
    bp = jnp.pad(b_pred, (0, L - 1))
    out16, _, _ = _sc_call(v0, edge_index[0], edge_index[1], bp)
    return out16[:1]


# speedup vs baseline: 100.7231x; 2.3422x over previous
import functools

import jax
import jax.numpy as jnp
from jax import lax
from jax.experimental import pallas as pl
from jax.experimental.pallas import tpu as pltpu
from jax.experimental.pallas import tpu_sc as plsc

N = 10000
E = 320000
D_IN = 128
D_ENC = 128
RB = 2048
NB = 5
NP = 10240
NT = 16
NPT = NP // NT
L = 16

EC_BIG = 157 * 128
EC_SMALL = 156 * 128


def _tc_body(x_ref, w_ref, b_ref, wp_ref, o_ref):
    i = pl.program_id(0)
    h = jax.lax.dot_general(
        x_ref[...], w_ref[...], (((1,), (0,)), ((), ())),
        preferred_element_type=jnp.float32)
    h = jnp.maximum(h + b_ref[...], 0.0)
    v = jax.lax.dot_general(
        wp_ref[...], h, (((1,), (1,)), ((), ())),
        preferred_element_type=jnp.float32)
    row = i * RB + jax.lax.broadcasted_iota(jnp.int32, (1, 1, RB), 2)
    o_ref[...] = jnp.where(row < N, v.reshape(1, 1, RB), 0.0)


def _tc_v0(x, W_in, b2, wp2):
    return pl.pallas_call(
        _tc_body,
        grid=(NB,),
        in_specs=[
            pl.BlockSpec((RB, D_IN), lambda i: (i, 0)),
            pl.BlockSpec((D_IN, D_ENC), lambda i: (0, 0)),
            pl.BlockSpec((1, D_ENC), lambda i: (0, 0)),
            pl.BlockSpec((1, D_ENC), lambda i: (0, 0)),
        ],
        out_specs=pl.BlockSpec((1, 1, RB), lambda i: (i, 0, 0)),
        out_shape=jax.ShapeDtypeStruct((NB, 1, RB), jnp.float32),
    )(x, W_in, b2, wp2)


def _sc_body(v0_hbm, ei_hbm, bp_hbm,
             out_hbm, part_hbm, vcur_hbm,
             ed_loc, v_loc, acc_loc, inv_loc, vnew_loc,
             red_loc, sum_loc, b_loc, out_loc):
    c = lax.axis_index("c")
    s = lax.axis_index("s")

    @pl.when(c == 0)
    def _main():
        wid = s
        nbase = wid * NPT

        eoff = jnp.where(wid < 4, wid * EC_BIG,
                         4 * EC_BIG + (wid - 4) * EC_SMALL)
        eload = pl.multiple_of(jnp.minimum(eoff, E - EC_BIG), 128)
        estart = eoff - eload
        ecnt = jnp.where(wid < 4, EC_BIG, EC_SMALL)

        pltpu.sync_copy(ei_hbm.at[:, pl.ds(eload, EC_BIG)], ed_loc)
        pltpu.sync_copy(v0_hbm, v_loc)
        pltpu.sync_copy(bp_hbm, b_loc)

        zeros16 = jnp.zeros((L,), jnp.float32)
        ones16 = jnp.ones((L,), jnp.float32)

        def zero_acc():
            @plsc.parallel_loop(0, NP, L, unroll=8)
            def _zb(i):
                acc_loc[pl.ds(i, L)] = zeros16

        def edge_pass(count_only):
            @plsc.parallel_loop(estart, estart + ecnt, L, unroll=8)
            def _eb(i):
                d16 = ed_loc[1, pl.ds(i, L)]
                if count_only:
                    vals = ones16
                else:
                    s16 = ed_loc[0, pl.ds(i, L)]
                    vals = plsc.load_gather(v_loc, [s16])
                plsc.addupdate_scatter(acc_loc, [d16], vals)

        def combine():
            pltpu.sync_copy(acc_loc, part_hbm.at[wid])
            plsc.subcore_barrier()
            pltpu.sync_copy(part_hbm.at[:, pl.ds(nbase, NPT)], red_loc)

        def reduced(jb):
            tot = red_loc[0, pl.ds(jb, L)]
            for k in range(1, NT):
                tot = tot + red_loc[k, pl.ds(jb, L)]
            return tot

        zero_acc()
        edge_pass(True)
        combine()

        @plsc.parallel_loop(0, NPT, L, unroll=4)
        def _invb(jb):
            inv_loc[pl.ds(jb, L)] = 1.0 / jnp.maximum(reduced(jb), 1.0)

        plsc.subcore_barrier()

        for step in range(2):
            zero_acc()
            edge_pass(False)
            combine()

            @plsc.parallel_loop(0, NPT, L, unroll=4)
            def _upd(jb):
                m = reduced(jb) * inv_loc[pl.ds(jb, L)]
                vold = v_loc[pl.ds(pl.multiple_of(nbase + jb, L), L)]
                vnew_loc[pl.ds(jb, L)] = (vold + m) * 0.5

            if step == 0:
                pltpu.sync_copy(vnew_loc, vcur_hbm.at[pl.ds(nbase, NPT)])
                plsc.subcore_barrier()
                pltpu.sync_copy(vcur_hbm, v_loc)
                plsc.subcore_barrier()

        @plsc.parallel_loop(0, NPT, L, unroll=4, carry=zeros16)
        def acc16(jb, a):
            return a + vnew_loc[pl.ds(jb, L)]

        plsc.subcore_barrier()
        out_loc[...] = acc16
        pltpu.sync_copy(out_loc, part_hbm.at[0, pl.ds(pl.multiple_of(wid * L, L), L)])
        plsc.subcore_barrier()

        @pl.when(s == 0)
        def _fin():
            pltpu.sync_copy(part_hbm.at[0, pl.ds(0, NT * L)], sum_loc)
            t = sum_loc[pl.ds(0, L)]
            for k in range(1, NT):
                t = t + sum_loc[pl.ds(k * L, L)]
            o = jnp.sum(t) * (1.0 / float(N))
            lane = lax.iota(jnp.int32, L)
            out_loc[...] = jnp.where(lane == 0, o, 0.0) + b_loc[...]
            pltpu.sync_copy(out_loc, out_hbm)


_sc_call = functools.partial(
    pl.kernel,
    mesh=plsc.VectorSubcoreMesh(core_axis_name="c", subcore_axis_name="s"),
    compiler_params=pltpu.CompilerParams(needs_layout_passes=False),
    out_type=(
        jax.ShapeDtypeStruct((L,), jnp.float32),
        jax.ShapeDtypeStruct((NT, NP), jnp.float32),
        jax.ShapeDtypeStruct((NP,), jnp.float32),
    ),
    scratch_types=[
        pltpu.VMEM((2, EC_BIG), jnp.int32),
        pltpu.VMEM((NP,), jnp.float32),
        pltpu.VMEM((NP,), jnp.float32),
        pltpu.VMEM((NPT,), jnp.float32),
        pltpu.VMEM((NPT,), jnp.float32),
        pltpu.VMEM((NT, NPT), jnp.float32),
        pltpu.VMEM((NT * L,), jnp.float32),
        pltpu.VMEM((L,), jnp.float32),
        pltpu.VMEM((L,), jnp.float32),
    ],
)(_sc_body)


def kernel(x, edge_index, W_in, b_in, W_pred, b_pred):
    b2 = b_in.reshape(1, D_ENC)
    wp2 = W_pred[:D_ENC, 0].reshape(1, D_ENC)
    v0 = _tc_v0(x, W_in, b2, wp2).reshape(NP)
    bp = jnp.pad(b_pred, (0, L - 1))
    out16, _, _ = _sc_call(v0, edge_index, bp)
    return out16[:1]
